# baseline (speedup 1.0000x reference)
import jax
import jax.numpy as jnp
from jax.experimental import pallas as pl
from jax.experimental.pallas import tpu as pltpu

_CHUNK = 4096


def _film_kernel(ids_ref, feat_ref, tableT_ref, wsT_ref, bs_ref, wbT_ref,
                 bb_ref, out_ref, s_ref, b_ref):
    bi = pl.program_id(0)
    hi = pl.program_id(1)

    @pl.when(hi == 0)
    def _compute_scale_shift():
        vid = ids_ref[bi]
        nv = tableT_ref.shape[1]
        onehot = (jax.lax.broadcasted_iota(jnp.int32, (nv, 1), 0)
                  == vid).astype(jnp.float32)
        emb = jnp.dot(tableT_ref[...], onehot,
                      preferred_element_type=jnp.float32)
        s = jnp.tanh(jnp.dot(wsT_ref[...], emb,
                             preferred_element_type=jnp.float32) + bs_ref[...])
        b = jnp.tanh(jnp.dot(wbT_ref[...], emb,
                             preferred_element_type=jnp.float32) + bb_ref[...])
        s_ref[...] = 1.0 + s
        b_ref[...] = b

    out_ref[...] = feat_ref[...] * s_ref[...][None] + b_ref[...][None]


def kernel(feat, vessel_ids, embed_table, Ws, bs, Wb, bb):
    B, CH, H, W = feat.shape
    NV = embed_table.shape[0]
    HW = H * W
    chunk = min(_CHUNK, HW)
    nchunk = HW // chunk

    feat3 = feat.reshape(B, CH, HW)
    ids = vessel_ids.astype(jnp.int32)
    tableT = embed_table.T
    wsT = Ws.T
    wbT = Wb.T
    bs_col = bs[:, None]
    bb_col = bb[:, None]

    grid_spec = pltpu.PrefetchScalarGridSpec(
        num_scalar_prefetch=1,
        grid=(B, nchunk),
        in_specs=[
            pl.BlockSpec((1, CH, chunk), lambda b, h, ids: (b, 0, h)),
            pl.BlockSpec((CH, NV), lambda b, h, ids: (0, 0)),
            pl.BlockSpec((CH, CH), lambda b, h, ids: (0, 0)),
            pl.BlockSpec((CH, 1), lambda b, h, ids: (0, 0)),
            pl.BlockSpec((CH, CH), lambda b, h, ids: (0, 0)),
            pl.BlockSpec((CH, 1), lambda b, h, ids: (0, 0)),
        ],
        out_specs=pl.BlockSpec((1, CH, chunk), lambda b, h, ids: (b, 0, h)),
        scratch_shapes=[
            pltpu.VMEM((CH, 1), jnp.float32),
            pltpu.VMEM((CH, 1), jnp.float32),
        ],
    )

    out3 = pl.pallas_call(
        _film_kernel,
        grid_spec=grid_spec,
        out_shape=jax.ShapeDtypeStruct((B, CH, HW), jnp.float32),
    )(ids, feat3, tableT, wsT, bs_col, wbT, bb_col)
    return out3.reshape(B, CH, H, W)


# speedup vs baseline: 3.4340x; 3.4340x over previous
import jax
import jax.numpy as jnp
from jax.experimental import pallas as pl
from jax.experimental.pallas import tpu as pltpu

_CH_BLK = 32


def _film_kernel(ids_ref, feat_ref, tableT_ref, wsT_ref, bs_ref, wbT_ref,
                 bb_ref, out_ref, s_ref, b_ref):
    bi = pl.program_id(0)
    ci = pl.program_id(1)

    @pl.when(ci == 0)
    def _compute_scale_shift():
        vid = ids_ref[bi]
        nv = tableT_ref.shape[1]
        onehot = (jax.lax.broadcasted_iota(jnp.int32, (nv, 1), 0)
                  == vid).astype(jnp.float32)
        emb = jnp.dot(tableT_ref[...], onehot,
                      preferred_element_type=jnp.float32)
        s = jnp.tanh(jnp.dot(wsT_ref[...], emb,
                             preferred_element_type=jnp.float32) + bs_ref[...])
        b = jnp.tanh(jnp.dot(wbT_ref[...], emb,
                             preferred_element_type=jnp.float32) + bb_ref[...])
        s_ref[...] = 1.0 + s
        b_ref[...] = b

    sblk = s_ref[pl.ds(ci * _CH_BLK, _CH_BLK), 0]
    bblk = b_ref[pl.ds(ci * _CH_BLK, _CH_BLK), 0]
    out_ref[...] = (feat_ref[...] * sblk[None, :, None, None]
                    + bblk[None, :, None, None])


def kernel(feat, vessel_ids, embed_table, Ws, bs, Wb, bb):
    B, CH, H, W = feat.shape
    NV = embed_table.shape[0]
    nc = CH // _CH_BLK

    ids = vessel_ids.astype(jnp.int32)
    tableT = embed_table.T
    wsT = Ws.T
    wbT = Wb.T
    bs_col = bs[:, None]
    bb_col = bb[:, None]

    grid_spec = pltpu.PrefetchScalarGridSpec(
        num_scalar_prefetch=1,
        grid=(B, nc),
        in_specs=[
            pl.BlockSpec((1, _CH_BLK, H, W), lambda b, c, ids: (b, c, 0, 0)),
            pl.BlockSpec((CH, NV), lambda b, c, ids: (0, 0)),
            pl.BlockSpec((CH, CH), lambda b, c, ids: (0, 0)),
            pl.BlockSpec((CH, 1), lambda b, c, ids: (0, 0)),
            pl.BlockSpec((CH, CH), lambda b, c, ids: (0, 0)),
            pl.BlockSpec((CH, 1), lambda b, c, ids: (0, 0)),
        ],
        out_specs=pl.BlockSpec((1, _CH_BLK, H, W), lambda b, c, ids: (b, c, 0, 0)),
        scratch_shapes=[
            pltpu.VMEM((CH, 1), jnp.float32),
            pltpu.VMEM((CH, 1), jnp.float32),
        ],
    )

    return pl.pallas_call(
        _film_kernel,
        grid_spec=grid_spec,
        out_shape=jax.ShapeDtypeStruct((B, CH, H, W), jnp.float32),
    )(ids, feat, tableT, wsT, bs_col, wbT, bb_col)


# speedup vs baseline: 3.9380x; 1.1467x over previous
import jax
import jax.numpy as jnp
from jax.experimental import pallas as pl
from jax.experimental.pallas import tpu as pltpu

_CH_BLK = 96


def _film_kernel(ids_ref, feat_ref, tableT_ref, wsT_ref, bs_ref, wbT_ref,
                 bb_ref, out_ref, s_ref, b_ref):
    bi = pl.program_id(0)
    ci = pl.program_id(1)

    @pl.when(ci == 0)
    def _compute_scale_shift():
        vid = ids_ref[bi]
        nv = tableT_ref.shape[1]
        onehot = (jax.lax.broadcasted_iota(jnp.int32, (nv, 1), 0)
                  == vid).astype(jnp.float32)
        emb = jnp.dot(tableT_ref[...], onehot,
                      preferred_element_type=jnp.float32)
        s = jnp.tanh(jnp.dot(wsT_ref[...], emb,
                             preferred_element_type=jnp.float32) + bs_ref[...])
        b = jnp.tanh(jnp.dot(wbT_ref[...], emb,
                             preferred_element_type=jnp.float32) + bb_ref[...])
        s_ref[...] = 1.0 + s
        b_ref[...] = b

    sblk = s_ref[pl.ds(ci * _CH_BLK, _CH_BLK), 0]
    bblk = b_ref[pl.ds(ci * _CH_BLK, _CH_BLK), 0]
    out_ref[...] = (feat_ref[...] * sblk[None, :, None, None]
                    + bblk[None, :, None, None])


def kernel(feat, vessel_ids, embed_table, Ws, bs, Wb, bb):
    B, CH, H, W = feat.shape
    NV = embed_table.shape[0]
    nc = CH // _CH_BLK

    ids = vessel_ids.astype(jnp.int32)
    tableT = embed_table.T
    wsT = Ws.T
    wbT = Wb.T
    bs_col = bs[:, None]
    bb_col = bb[:, None]

    grid_spec = pltpu.PrefetchScalarGridSpec(
        num_scalar_prefetch=1,
        grid=(B, nc),
        in_specs=[
            pl.BlockSpec((1, _CH_BLK, H, W), lambda b, c, ids: (b, c, 0, 0)),
            pl.BlockSpec((CH, NV), lambda b, c, ids: (0, 0)),
            pl.BlockSpec((CH, CH), lambda b, c, ids: (0, 0)),
            pl.BlockSpec((CH, 1), lambda b, c, ids: (0, 0)),
            pl.BlockSpec((CH, CH), lambda b, c, ids: (0, 0)),
            pl.BlockSpec((CH, 1), lambda b, c, ids: (0, 0)),
        ],
        out_specs=pl.BlockSpec((1, _CH_BLK, H, W), lambda b, c, ids: (b, c, 0, 0)),
        scratch_shapes=[
            pltpu.VMEM((CH, 1), jnp.float32),
            pltpu.VMEM((CH, 1), jnp.float32),
        ],
    )

    return pl.pallas_call(
        _film_kernel,
        grid_spec=grid_spec,
        out_shape=jax.ShapeDtypeStruct((B, CH, H, W), jnp.float32),
    )(ids, feat, tableT, wsT, bs_col, wbT, bb_col)


# speedup vs baseline: 4.0531x; 1.0292x over previous
import jax
import jax.numpy as jnp
from jax.experimental import pallas as pl
from jax.experimental.pallas import tpu as pltpu

_BB = 2


def _film_kernel(ids_ref, feat_ref, tableT_ref, wsT_ref, bs_ref, wbT_ref,
                 bb_ref, out_ref):
    g = pl.program_id(0)
    ch = tableT_ref.shape[0]
    nv = tableT_ref.shape[1]
    for j in range(_BB):
        vid = ids_ref[g * _BB + j]
        onehot = (jax.lax.broadcasted_iota(jnp.int32, (nv, 1), 0)
                  == vid).astype(jnp.float32)
        emb = jnp.dot(tableT_ref[...], onehot,
                      preferred_element_type=jnp.float32)
        s = jnp.tanh(jnp.dot(wsT_ref[...], emb,
                             preferred_element_type=jnp.float32) + bs_ref[...])
        b = jnp.tanh(jnp.dot(wbT_ref[...], emb,
                             preferred_element_type=jnp.float32) + bb_ref[...])
        s4 = (1.0 + s).reshape(1, ch, 1, 1)
        b4 = b.reshape(1, ch, 1, 1)
        out_ref[pl.ds(j, 1)] = feat_ref[pl.ds(j, 1)] * s4 + b4


def kernel(feat, vessel_ids, embed_table, Ws, bs, Wb, bb):
    B, CH, H, W = feat.shape
    NV = embed_table.shape[0]
    ng = B // _BB

    ids = vessel_ids.astype(jnp.int32)
    tableT = embed_table.T
    wsT = Ws.T
    wbT = Wb.T
    bs_col = bs[:, None]
    bb_col = bb[:, None]

    grid_spec = pltpu.PrefetchScalarGridSpec(
        num_scalar_prefetch=1,
        grid=(ng,),
        in_specs=[
            pl.BlockSpec((_BB, CH, H, W), lambda g, ids: (g, 0, 0, 0)),
            pl.BlockSpec((CH, NV), lambda g, ids: (0, 0)),
            pl.BlockSpec((CH, CH), lambda g, ids: (0, 0)),
            pl.BlockSpec((CH, 1), lambda g, ids: (0, 0)),
            pl.BlockSpec((CH, CH), lambda g, ids: (0, 0)),
            pl.BlockSpec((CH, 1), lambda g, ids: (0, 0)),
        ],
        out_specs=pl.BlockSpec((_BB, CH, H, W), lambda g, ids: (g, 0, 0, 0)),
    )

    return pl.pallas_call(
        _film_kernel,
        grid_spec=grid_spec,
        out_shape=jax.ShapeDtypeStruct((B, CH, H, W), jnp.float32),
    )(ids, feat, tableT, wsT, bs_col, wbT, bb_col)
